# baseline (speedup 1.0000x reference)
import functools

import jax
import jax.numpy as jnp
from jax import lax
from jax.experimental import pallas as pl
from jax.experimental.pallas import tpu as pltpu
from jax.experimental.pallas import tpu_sc as plsc

_PTS_PAD = 16


def _sc_gather(feat_tbl, pts_tbl, idx_flat, chunk):
    E = idx_flat.shape[0]
    C = feat_tbl.shape[1]
    info = plsc.get_sparse_core_info()
    ncores, nsub = info.num_cores, info.num_subcores
    nw = ncores * nsub
    per_w = E // nw
    n_chunks = per_w // chunk

    mesh = plsc.VectorSubcoreMesh(core_axis_name="c", subcore_axis_name="s")

    @functools.partial(
        pl.kernel,
        mesh=mesh,
        compiler_params=pltpu.CompilerParams(use_tc_tiling_on_sc=False),
        out_type=[
            jax.ShapeDtypeStruct((E, C), jnp.bfloat16),
            jax.ShapeDtypeStruct((E, _PTS_PAD), jnp.float32),
        ],
        scratch_types=[
            pltpu.VMEM((chunk,), jnp.int32),
            pltpu.VMEM((chunk, C), jnp.bfloat16),
            pltpu.VMEM((chunk, _PTS_PAD), jnp.float32),
            pltpu.SemaphoreType.DMA,
            pltpu.SemaphoreType.DMA,
        ],
    )
    def gather_kernel(feat_hbm, pts_hbm, idx_hbm, feat_out, pts_out,
                      idx_v, feat_v, pts_v, sem_f, sem_p):
        wid = lax.axis_index("s") * ncores + lax.axis_index("c")
        base = wid * per_w

        def body(i, carry):
            off = base + i * chunk
            pltpu.sync_copy(idx_hbm.at[pl.ds(off, chunk)], idx_v)
            cp_f = pltpu.async_copy(feat_hbm.at[idx_v], feat_v, sem_f)
            cp_p = pltpu.async_copy(pts_hbm.at[idx_v], pts_v, sem_p)
            cp_f.wait()
            cp_p.wait()
            pltpu.sync_copy(feat_v, feat_out.at[pl.ds(off, chunk)])
            pltpu.sync_copy(pts_v, pts_out.at[pl.ds(off, chunk)])
            return carry

        lax.fori_loop(0, n_chunks, body, 0)

    return gather_kernel(feat_tbl, pts_tbl, idx_flat)


def _tc_body(featg_ref, ptsg_ref, nxt_ref, w1_ref, b1_ref, w2_ref, b2_ref,
             w3_ref, b3_ref, wn_ref, bias_ref, out_ref, *, kk, nc):
    p = nxt_ref.shape[0]
    c = featg_ref.shape[1]
    rel = ptsg_ref[...].reshape(p, kk, _PTS_PAD) - nxt_ref[...][:, None, :]
    rel = rel.reshape(p * kk, _PTS_PAD)
    h = jnp.dot(rel, w1_ref[...], preferred_element_type=jnp.float32)
    h = jnp.maximum(h + b1_ref[...], 0.0)
    h = jnp.dot(h, w2_ref[...], preferred_element_type=jnp.float32)
    h = jnp.maximum(h + b2_ref[...], 0.0)
    d = jnp.dot(h, w3_ref[...], preferred_element_type=jnp.float32)
    d = jnp.maximum(d + b3_ref[...], 0.0)
    d3 = d.astype(jnp.bfloat16).reshape(p, kk, nc)
    feat3 = featg_ref[...].reshape(p, kk, c)
    fpre = lax.dot_general(d3, feat3, (((1,), (1,)), ((0,), (0,))),
                           preferred_element_type=jnp.float32)
    out_ref[...] = jnp.dot(fpre.reshape(p, nc * c), wn_ref[...],
                           preferred_element_type=jnp.float32) + bias_ref[...]


def kernel(inp, points, next_pts, indices_, K, weight, bias, centers,
           w1, b1, w2, b2, w3, b3):
    B, N, C_IN = inp.shape
    DIM = points.shape[2]
    NC = centers.shape[1]
    C_OUT = weight.shape[2]
    K = indices_.shape[2]
    E = B * N * K

    offs = (jnp.arange(B, dtype=jnp.int32) * N)[:, None, None]
    idx_flat = (indices_.astype(jnp.int32) + offs).reshape(E)
    feat_tbl = inp.reshape(B * N, C_IN).astype(jnp.bfloat16)
    pts_tbl = jnp.pad(points.reshape(B * N, DIM), ((0, 0), (0, _PTS_PAD - DIM)))
    nxt_flat = jnp.pad(next_pts.reshape(B * N, DIM),
                       ((0, 0), (0, _PTS_PAD - DIM)))

    w1r = w1.reshape(2 * NC, DIM, NC)
    w1e = jnp.sum(w1r, axis=2).T
    w1p = jnp.pad(w1e, ((0, _PTS_PAD - DIM), (0, 0)))
    b1e = (b1 - jnp.sum(w1r * centers[None], axis=(1, 2))).reshape(1, 2 * NC)
    w2t = w2.T
    b2r = b2.reshape(1, NC)
    w3t = w3.T
    b3r = b3.reshape(1, NC)
    wn = (jnp.transpose(weight, (1, 0, 2)) / K).reshape(NC * C_IN, C_OUT)
    bias_r = bias.reshape(1, C_OUT)

    featg, ptsg = _sc_gather(feat_tbl, pts_tbl, idx_flat, chunk=1024)

    P = 512
    nb = (B * N) // P
    body = functools.partial(_tc_body, kk=K, nc=NC)
    out = pl.pallas_call(
        body,
        grid=(nb,),
        in_specs=[
            pl.BlockSpec((P * K, C_IN), lambda i: (i, 0)),
            pl.BlockSpec((P * K, _PTS_PAD), lambda i: (i, 0)),
            pl.BlockSpec((P, _PTS_PAD), lambda i: (i, 0)),
            pl.BlockSpec((_PTS_PAD, 2 * NC), lambda i: (0, 0)),
            pl.BlockSpec((1, 2 * NC), lambda i: (0, 0)),
            pl.BlockSpec((2 * NC, NC), lambda i: (0, 0)),
            pl.BlockSpec((1, NC), lambda i: (0, 0)),
            pl.BlockSpec((NC, NC), lambda i: (0, 0)),
            pl.BlockSpec((1, NC), lambda i: (0, 0)),
            pl.BlockSpec((C_IN * NC, C_OUT), lambda i: (0, 0)),
            pl.BlockSpec((1, C_OUT), lambda i: (0, 0)),
        ],
        out_specs=pl.BlockSpec((P, C_OUT), lambda i: (i, 0)),
        out_shape=jax.ShapeDtypeStruct((B * N, C_OUT), jnp.float32),
    )(featg, ptsg, nxt_flat, w1p, b1e, w2t, b2r, w3t, b3r, wn, bias_r)

    return out.reshape(B, N, C_OUT)


# speedup vs baseline: 1.1113x; 1.1113x over previous
import functools

import jax
import jax.numpy as jnp
from jax import lax
from jax.experimental import pallas as pl
from jax.experimental.pallas import tpu as pltpu
from jax.experimental.pallas import tpu_sc as plsc

_PTS_PAD = 16


def _sc_gather(feat_tbl, pts_tbl, idx_flat, chunk):
    E = idx_flat.shape[0]
    C = feat_tbl.shape[1]
    info = plsc.get_sparse_core_info()
    ncores, nsub = info.num_cores, info.num_subcores
    nw = ncores * nsub
    per_w = E // nw
    n_chunks = per_w // chunk

    mesh = plsc.VectorSubcoreMesh(core_axis_name="c", subcore_axis_name="s")

    @functools.partial(
        pl.kernel,
        mesh=mesh,
        compiler_params=pltpu.CompilerParams(use_tc_tiling_on_sc=False),
        out_type=[
            jax.ShapeDtypeStruct((E, C), jnp.float32),
            jax.ShapeDtypeStruct((E, _PTS_PAD), jnp.float32),
        ],
        scratch_types=[
            pltpu.VMEM((chunk,), jnp.int32),
            pltpu.VMEM((chunk, C), jnp.float32),
            pltpu.VMEM((chunk, _PTS_PAD), jnp.float32),
            pltpu.SemaphoreType.DMA,
            pltpu.SemaphoreType.DMA,
        ],
    )
    def gather_kernel(feat_hbm, pts_hbm, idx_hbm, feat_out, pts_out,
                      idx_v, feat_v, pts_v, sem_f, sem_p):
        wid = lax.axis_index("s") * ncores + lax.axis_index("c")
        base = wid * per_w

        def body(i, carry):
            off = base + i * chunk
            pltpu.sync_copy(idx_hbm.at[pl.ds(off, chunk)], idx_v)
            cp_f = pltpu.async_copy(feat_hbm.at[idx_v], feat_v, sem_f)
            cp_p = pltpu.async_copy(pts_hbm.at[idx_v], pts_v, sem_p)
            cp_f.wait()
            cp_p.wait()
            pltpu.sync_copy(feat_v, feat_out.at[pl.ds(off, chunk)])
            pltpu.sync_copy(pts_v, pts_out.at[pl.ds(off, chunk)])
            return carry

        lax.fori_loop(0, n_chunks, body, 0)

    return gather_kernel(feat_tbl, pts_tbl, idx_flat)


def _tc_body(featg_ref, ptsg_ref, nxt_ref, w1_ref, b1_ref, w2_ref, b2_ref,
             w3_ref, b3_ref, wn_ref, bias_ref, out_ref, *, kk, nc):
    p = nxt_ref.shape[0]
    c = featg_ref.shape[1]
    rel = ptsg_ref[...].reshape(p, kk, _PTS_PAD) - nxt_ref[...][:, None, :]
    rel = rel.reshape(p * kk, _PTS_PAD)
    h = jnp.dot(rel, w1_ref[...], preferred_element_type=jnp.float32)
    h = jnp.maximum(h + b1_ref[...], 0.0)
    h = jnp.dot(h, w2_ref[...], preferred_element_type=jnp.float32)
    h = jnp.maximum(h + b2_ref[...], 0.0)
    d = jnp.dot(h, w3_ref[...], preferred_element_type=jnp.float32)
    d = jnp.maximum(d + b3_ref[...], 0.0)
    d3 = d.reshape(p, kk, nc)
    feat3 = featg_ref[...].reshape(p, kk, c)
    fpre = lax.dot_general(d3, feat3, (((1,), (1,)), ((0,), (0,))),
                           preferred_element_type=jnp.float32)
    out_ref[...] = jnp.dot(fpre.reshape(p, nc * c), wn_ref[...],
                           preferred_element_type=jnp.float32) + bias_ref[...]


def kernel(inp, points, next_pts, indices_, K, weight, bias, centers,
           w1, b1, w2, b2, w3, b3):
    B, N, C_IN = inp.shape
    DIM = points.shape[2]
    NC = centers.shape[1]
    C_OUT = weight.shape[2]
    K = indices_.shape[2]
    E = B * N * K

    offs = (jnp.arange(B, dtype=jnp.int32) * N)[:, None, None]
    idx_flat = (indices_.astype(jnp.int32) + offs).reshape(E)
    feat_tbl = inp.reshape(B * N, C_IN)
    pts_tbl = jnp.pad(points.reshape(B * N, DIM), ((0, 0), (0, _PTS_PAD - DIM)))
    nxt_flat = jnp.pad(next_pts.reshape(B * N, DIM),
                       ((0, 0), (0, _PTS_PAD - DIM)))

    w1r = w1.reshape(2 * NC, DIM, NC)
    w1e = jnp.sum(w1r, axis=2).T
    w1p = jnp.pad(w1e, ((0, _PTS_PAD - DIM), (0, 0)))
    b1e = (b1 - jnp.sum(w1r * centers[None], axis=(1, 2))).reshape(1, 2 * NC)
    w2t = w2.T
    b2r = b2.reshape(1, NC)
    w3t = w3.T
    b3r = b3.reshape(1, NC)
    wn = (jnp.transpose(weight, (1, 0, 2)) / K).reshape(NC * C_IN, C_OUT)
    bias_r = bias.reshape(1, C_OUT)

    NCH = 4
    P = 512
    pts_per_chunk = (B * N) // NCH
    e_per_chunk = pts_per_chunk * K
    nb = pts_per_chunk // P
    body = functools.partial(_tc_body, kk=K, nc=NC)
    tc_call = pl.pallas_call(
        body,
        grid=(nb,),
        in_specs=[
            pl.BlockSpec((P * K, C_IN), lambda i: (i, 0)),
            pl.BlockSpec((P * K, _PTS_PAD), lambda i: (i, 0)),
            pl.BlockSpec((P, _PTS_PAD), lambda i: (i, 0)),
            pl.BlockSpec((_PTS_PAD, 2 * NC), lambda i: (0, 0)),
            pl.BlockSpec((1, 2 * NC), lambda i: (0, 0)),
            pl.BlockSpec((2 * NC, NC), lambda i: (0, 0)),
            pl.BlockSpec((1, NC), lambda i: (0, 0)),
            pl.BlockSpec((NC, NC), lambda i: (0, 0)),
            pl.BlockSpec((1, NC), lambda i: (0, 0)),
            pl.BlockSpec((C_IN * NC, C_OUT), lambda i: (0, 0)),
            pl.BlockSpec((1, C_OUT), lambda i: (0, 0)),
        ],
        out_specs=pl.BlockSpec((P, C_OUT), lambda i: (i, 0)),
        out_shape=jax.ShapeDtypeStruct((pts_per_chunk, C_OUT), jnp.float32),
    )
    outs = []
    for ci in range(NCH):
        idx_c = lax.slice(idx_flat, (ci * e_per_chunk,),
                          ((ci + 1) * e_per_chunk,))
        featg, ptsg = _sc_gather(feat_tbl, pts_tbl, idx_c, chunk=1024)
        nxt_c = lax.slice(nxt_flat, (ci * pts_per_chunk, 0),
                          ((ci + 1) * pts_per_chunk, _PTS_PAD))
        outs.append(tc_call(featg, ptsg, nxt_c, w1p, b1e, w2t, b2r,
                            w3t, b3r, wn, bias_r))
    out = jnp.concatenate(outs, axis=0)

    return out.reshape(B, N, C_OUT)


# speedup vs baseline: 2.0889x; 1.8796x over previous
import functools

import jax
import jax.numpy as jnp
from jax import lax
from jax.experimental import pallas as pl
from jax.experimental.pallas import tpu as pltpu
from jax.experimental.pallas import tpu_sc as plsc

_W = 128


def _sc_gather(tbl, idx_full, e0, e_chunk, chunk):
    info = plsc.get_sparse_core_info()
    ncores, nsub = info.num_cores, info.num_subcores
    nw = ncores * nsub
    per_w = e_chunk // nw
    n_chunks = per_w // chunk

    mesh = plsc.VectorSubcoreMesh(core_axis_name="c", subcore_axis_name="s")

    @functools.partial(
        pl.kernel,
        mesh=mesh,
        compiler_params=pltpu.CompilerParams(use_tc_tiling_on_sc=True),
        out_type=jax.ShapeDtypeStruct((e_chunk, _W), jnp.float32),
        scratch_types=[
            pltpu.VMEM((chunk,), jnp.int32),
            pltpu.VMEM((chunk, _W), jnp.float32),
            pltpu.SemaphoreType.DMA,
        ],
    )
    def gather_kernel(tbl_hbm, idx_hbm, out_hbm, idx_v, row_v, sem):
        wid = lax.axis_index("s") * ncores + lax.axis_index("c")
        base = wid * per_w

        def body(i, carry):
            off = base + i * chunk
            pltpu.sync_copy(idx_hbm.at[pl.ds(e0 + off, chunk)], idx_v)
            pltpu.async_copy(tbl_hbm.at[idx_v], row_v, sem).wait()
            pltpu.sync_copy(row_v, out_hbm.at[pl.ds(off, chunk)])
            return carry

        lax.fori_loop(0, n_chunks, body, 0)

    return gather_kernel(tbl, idx_full)


def _tc_body(g_ref, nxt_ref, w1_ref, b1_ref, w2_ref, b2_ref,
             w3_ref, b3_ref, wn_ref, bias_ref, out_ref, *, kk, nc, c):
    p = nxt_ref.shape[0]
    g = g_ref[...]
    g3 = g.reshape(p, kk, _W)
    rel = (g3 - nxt_ref[...][:, None, :]).reshape(p * kk, _W)
    h = jnp.dot(rel, w1_ref[...], preferred_element_type=jnp.float32)
    h = jnp.maximum(h + b1_ref[...], 0.0)
    h = jnp.dot(h, w2_ref[...], preferred_element_type=jnp.float32)
    h = jnp.maximum(h + b2_ref[...], 0.0)
    d = jnp.dot(h, w3_ref[...], preferred_element_type=jnp.float32)
    d = jnp.maximum(d + b3_ref[...], 0.0)
    d3 = d.reshape(p, kk, nc)
    feat3 = g3[:, :, :c]
    fpre = lax.dot_general(d3, feat3, (((1,), (1,)), ((0,), (0,))),
                           preferred_element_type=jnp.float32)
    out_ref[...] = jnp.dot(fpre.reshape(p, nc * c), wn_ref[...],
                           preferred_element_type=jnp.float32) + bias_ref[...]


def kernel(inp, points, next_pts, indices_, K, weight, bias, centers,
           w1, b1, w2, b2, w3, b3):
    B, N, C_IN = inp.shape
    DIM = points.shape[2]
    NC = centers.shape[1]
    C_OUT = weight.shape[2]
    K = indices_.shape[2]
    E = B * N * K

    offs = (jnp.arange(B, dtype=jnp.int32) * N)[:, None, None]
    idx_flat = (indices_.astype(jnp.int32) + offs).reshape(E)
    tbl = jnp.pad(
        jnp.concatenate([inp.reshape(B * N, C_IN),
                         points.reshape(B * N, DIM)], axis=1),
        ((0, 0), (0, _W - C_IN - DIM)))
    nxt_flat = jnp.pad(next_pts.reshape(B * N, DIM),
                       ((0, 0), (C_IN, _W - C_IN - DIM)))

    w1r = w1.reshape(2 * NC, DIM, NC)
    w1e = jnp.sum(w1r, axis=2).T
    w1p = jnp.pad(w1e, ((C_IN, _W - C_IN - DIM), (0, 0)))
    b1e = (b1 - jnp.sum(w1r * centers[None], axis=(1, 2))).reshape(1, 2 * NC)
    w2t = w2.T
    b2r = b2.reshape(1, NC)
    w3t = w3.T
    b3r = b3.reshape(1, NC)
    wn = (jnp.transpose(weight, (1, 0, 2)) / K).reshape(NC * C_IN, C_OUT)
    bias_r = bias.reshape(1, C_OUT)

    NCH = 4
    P = 512
    pts_per_chunk = (B * N) // NCH
    e_per_chunk = pts_per_chunk * K
    nb = pts_per_chunk // P
    body = functools.partial(_tc_body, kk=K, nc=NC, c=C_IN)
    outs = []
    for ci in range(NCH):
        g = _sc_gather(tbl, idx_flat, ci * e_per_chunk, e_per_chunk, 512)
        base = ci * nb
        tc_call = pl.pallas_call(
            body,
            grid=(nb,),
            in_specs=[
                pl.BlockSpec((P * K, _W), lambda i: (i, 0)),
                pl.BlockSpec((P, _W), lambda i, b=base: (b + i, 0)),
                pl.BlockSpec((_W, 2 * NC), lambda i: (0, 0)),
                pl.BlockSpec((1, 2 * NC), lambda i: (0, 0)),
                pl.BlockSpec((2 * NC, NC), lambda i: (0, 0)),
                pl.BlockSpec((1, NC), lambda i: (0, 0)),
                pl.BlockSpec((NC, NC), lambda i: (0, 0)),
                pl.BlockSpec((1, NC), lambda i: (0, 0)),
                pl.BlockSpec((C_IN * NC, C_OUT), lambda i: (0, 0)),
                pl.BlockSpec((1, C_OUT), lambda i: (0, 0)),
            ],
            out_specs=pl.BlockSpec((P, C_OUT), lambda i: (i, 0)),
            out_shape=jax.ShapeDtypeStruct((pts_per_chunk, C_OUT), jnp.float32),
        )
        outs.append(tc_call(g, nxt_flat, w1p, b1e, w2t, b2r,
                            w3t, b3r, wn, bias_r))
    out = jnp.concatenate(outs, axis=0)

    return out.reshape(B, N, C_OUT)


# speedup vs baseline: 2.1351x; 1.0221x over previous
import functools

import jax
import jax.numpy as jnp
from jax import lax
from jax.experimental import pallas as pl
from jax.experimental.pallas import tpu as pltpu
from jax.experimental.pallas import tpu_sc as plsc

_W = 128


def _sc_gather(tbl, idx_full, e0, e_chunk, chunk):
    info = plsc.get_sparse_core_info()
    ncores, nsub = info.num_cores, info.num_subcores
    nw = ncores * nsub
    per_w = e_chunk // nw
    n_pairs = per_w // (2 * chunk)

    mesh = plsc.VectorSubcoreMesh(core_axis_name="c", subcore_axis_name="s")

    @functools.partial(
        pl.kernel,
        mesh=mesh,
        compiler_params=pltpu.CompilerParams(use_tc_tiling_on_sc=True),
        out_type=jax.ShapeDtypeStruct((e_chunk, _W), jnp.float32),
        scratch_types=[
            pltpu.VMEM((chunk,), jnp.int32),
            pltpu.VMEM((chunk,), jnp.int32),
            pltpu.VMEM((chunk, _W), jnp.float32),
            pltpu.VMEM((chunk, _W), jnp.float32),
            pltpu.SemaphoreType.DMA,
            pltpu.SemaphoreType.DMA,
            pltpu.SemaphoreType.DMA,
            pltpu.SemaphoreType.DMA,
        ],
    )
    def gather_kernel(tbl_hbm, idx_hbm, out_hbm, idx_v0, idx_v1,
                      row_v0, row_v1, gsem0, gsem1, wsem0, wsem1):
        wid = lax.axis_index("s") * ncores + lax.axis_index("c")
        base = wid * per_w

        def body(i, carry):
            off0 = base + i * 2 * chunk
            off1 = off0 + chunk
            pltpu.sync_copy(idx_hbm.at[pl.ds(e0 + off0, chunk)], idx_v0)
            g0 = pltpu.async_copy(tbl_hbm.at[idx_v0], row_v0, gsem0)
            pltpu.sync_copy(idx_hbm.at[pl.ds(e0 + off1, chunk)], idx_v1)
            g1 = pltpu.async_copy(tbl_hbm.at[idx_v1], row_v1, gsem1)
            g0.wait()
            w0 = pltpu.async_copy(row_v0, out_hbm.at[pl.ds(off0, chunk)],
                                  wsem0)
            g1.wait()
            w1 = pltpu.async_copy(row_v1, out_hbm.at[pl.ds(off1, chunk)],
                                  wsem1)
            w0.wait()
            w1.wait()
            return carry

        lax.fori_loop(0, n_pairs, body, 0)

    return gather_kernel(tbl, idx_full)


def _tc_body(g_ref, hb_ref, w1_ref, w2_ref, b2_ref,
             w3_ref, b3_ref, wn_ref, bias_ref, out_ref, *, kk, nc, c):
    p = hb_ref.shape[0]
    g = g_ref[...]
    h = jnp.dot(g, w1_ref[...], preferred_element_type=jnp.float32)
    h = h.reshape(p, kk, 2 * nc) + hb_ref[...][:, None, :]
    h = jnp.maximum(h, 0.0).reshape(p * kk, 2 * nc)
    h = jnp.dot(h, w2_ref[...], preferred_element_type=jnp.float32)
    h = jnp.maximum(h + b2_ref[...], 0.0)
    d = jnp.dot(h, w3_ref[...], preferred_element_type=jnp.float32)
    d = jnp.maximum(d + b3_ref[...], 0.0)
    d3 = d.reshape(p, kk, nc)
    feat3 = g.reshape(p, kk, _W)[:, :, :c]
    fpre = lax.dot_general(d3, feat3, (((1,), (1,)), ((0,), (0,))),
                           preferred_element_type=jnp.float32)
    out_ref[...] = jnp.dot(fpre.reshape(p, nc * c), wn_ref[...],
                           preferred_element_type=jnp.float32) + bias_ref[...]


def kernel(inp, points, next_pts, indices_, K, weight, bias, centers,
           w1, b1, w2, b2, w3, b3):
    B, N, C_IN = inp.shape
    DIM = points.shape[2]
    NC = centers.shape[1]
    C_OUT = weight.shape[2]
    K = indices_.shape[2]
    E = B * N * K

    offs = (jnp.arange(B, dtype=jnp.int32) * N)[:, None, None]
    idx_flat = (indices_.astype(jnp.int32) + offs).reshape(E)
    tbl = jnp.pad(
        jnp.concatenate([inp.reshape(B * N, C_IN),
                         points.reshape(B * N, DIM)], axis=1),
        ((0, 0), (0, _W - C_IN - DIM)))

    w1r = w1.reshape(2 * NC, DIM, NC)
    w1e = jnp.sum(w1r, axis=2).T
    w1p = jnp.pad(w1e, ((C_IN, _W - C_IN - DIM), (0, 0)))
    b1e = b1 - jnp.sum(w1r * centers[None], axis=(1, 2))
    hb = b1e[None] - next_pts.reshape(B * N, DIM) @ w1e
    w2t = w2.T
    b2r = b2.reshape(1, NC)
    w3t = w3.T
    b3r = b3.reshape(1, NC)
    wn = (jnp.transpose(weight, (1, 0, 2)) / K).reshape(NC * C_IN, C_OUT)
    bias_r = bias.reshape(1, C_OUT)

    NCH = 4
    P = 512
    pts_per_chunk = (B * N) // NCH
    e_per_chunk = pts_per_chunk * K
    nb = pts_per_chunk // P
    body = functools.partial(_tc_body, kk=K, nc=NC, c=C_IN)
    outs = []
    for ci in range(NCH):
        g = _sc_gather(tbl, idx_flat, ci * e_per_chunk, e_per_chunk, 256)
        base = ci * nb
        tc_call = pl.pallas_call(
            body,
            grid=(nb,),
            in_specs=[
                pl.BlockSpec((P * K, _W), lambda i: (i, 0)),
                pl.BlockSpec((P, 2 * NC), lambda i, b=base: (b + i, 0)),
                pl.BlockSpec((_W, 2 * NC), lambda i: (0, 0)),
                pl.BlockSpec((2 * NC, NC), lambda i: (0, 0)),
                pl.BlockSpec((1, NC), lambda i: (0, 0)),
                pl.BlockSpec((NC, NC), lambda i: (0, 0)),
                pl.BlockSpec((1, NC), lambda i: (0, 0)),
                pl.BlockSpec((C_IN * NC, C_OUT), lambda i: (0, 0)),
                pl.BlockSpec((1, C_OUT), lambda i: (0, 0)),
            ],
            out_specs=pl.BlockSpec((P, C_OUT), lambda i: (i, 0)),
            out_shape=jax.ShapeDtypeStruct((pts_per_chunk, C_OUT), jnp.float32),
        )
        outs.append(tc_call(g, hb, w1p, w2t, b2r, w3t, b3r, wn, bias_r))
    out = jnp.concatenate(outs, axis=0)

    return out.reshape(B, N, C_OUT)


# speedup vs baseline: 2.1869x; 1.0243x over previous
import functools

import jax
import jax.numpy as jnp
from jax import lax
from jax.experimental import pallas as pl
from jax.experimental.pallas import tpu as pltpu
from jax.experimental.pallas import tpu_sc as plsc

_W = 128


def _sc_gather(tbl, idx_full, e0, e_chunk, chunk):
    info = plsc.get_sparse_core_info()
    ncores, nsub = info.num_cores, info.num_subcores
    nw = ncores * nsub
    per_w = e_chunk // nw
    n_pairs = per_w // (2 * chunk)

    mesh = plsc.VectorSubcoreMesh(core_axis_name="c", subcore_axis_name="s")

    @functools.partial(
        pl.kernel,
        mesh=mesh,
        compiler_params=pltpu.CompilerParams(use_tc_tiling_on_sc=True),
        out_type=jax.ShapeDtypeStruct((e_chunk, _W), jnp.float32),
        scratch_types=[
            pltpu.VMEM((chunk,), jnp.int32),
            pltpu.VMEM((chunk,), jnp.int32),
            pltpu.VMEM((chunk, _W), jnp.float32),
            pltpu.VMEM((chunk, _W), jnp.float32),
            pltpu.SemaphoreType.DMA,
            pltpu.SemaphoreType.DMA,
            pltpu.SemaphoreType.DMA,
            pltpu.SemaphoreType.DMA,
        ],
    )
    def gather_kernel(tbl_hbm, idx_hbm, out_hbm, idx_v0, idx_v1,
                      row_v0, row_v1, gsem0, gsem1, wsem0, wsem1):
        wid = lax.axis_index("s") * ncores + lax.axis_index("c")
        base = wid * per_w

        def body(i, carry):
            off0 = base + i * 2 * chunk
            off1 = off0 + chunk
            pltpu.sync_copy(idx_hbm.at[pl.ds(e0 + off0, chunk)], idx_v0)
            g0 = pltpu.async_copy(tbl_hbm.at[idx_v0], row_v0, gsem0)
            pltpu.sync_copy(idx_hbm.at[pl.ds(e0 + off1, chunk)], idx_v1)
            g1 = pltpu.async_copy(tbl_hbm.at[idx_v1], row_v1, gsem1)
            g0.wait()
            w0 = pltpu.async_copy(row_v0, out_hbm.at[pl.ds(off0, chunk)],
                                  wsem0)
            g1.wait()
            w1 = pltpu.async_copy(row_v1, out_hbm.at[pl.ds(off1, chunk)],
                                  wsem1)
            w0.wait()
            w1.wait()
            return carry

        lax.fori_loop(0, n_pairs, body, 0)

    return gather_kernel(tbl, idx_full)


def _tc_body(g_ref, hb_ref, w1_ref, w2_ref, b2_ref,
             w3_ref, b3_ref, wn_ref, bias_ref, out_ref, *, kk, nc, c):
    p = hb_ref.shape[0]
    g = g_ref[...]
    h = jnp.dot(g, w1_ref[...], preferred_element_type=jnp.float32)
    h = h.reshape(p, kk, 2 * nc) + hb_ref[...][:, None, :]
    h = jnp.maximum(h, 0.0).reshape(p * kk, 2 * nc)
    h = jnp.dot(h, w2_ref[...], preferred_element_type=jnp.float32)
    h = jnp.maximum(h + b2_ref[...], 0.0)
    d = jnp.dot(h, w3_ref[...], preferred_element_type=jnp.float32)
    d = jnp.maximum(d + b3_ref[...], 0.0)
    d3 = d.reshape(p, kk, nc)
    feat3 = g.reshape(p, kk, _W)[:, :, :c]
    fpre = lax.dot_general(d3, feat3, (((1,), (1,)), ((0,), (0,))),
                           preferred_element_type=jnp.float32)
    out_ref[...] = jnp.dot(fpre.reshape(p, nc * c), wn_ref[...],
                           preferred_element_type=jnp.float32) + bias_ref[...]


def kernel(inp, points, next_pts, indices_, K, weight, bias, centers,
           w1, b1, w2, b2, w3, b3):
    B, N, C_IN = inp.shape
    DIM = points.shape[2]
    NC = centers.shape[1]
    C_OUT = weight.shape[2]
    K = indices_.shape[2]
    E = B * N * K

    offs = (jnp.arange(B, dtype=jnp.int32) * N)[:, None, None]
    idx_flat = (indices_.astype(jnp.int32) + offs).reshape(E)
    tbl = jnp.pad(
        jnp.concatenate([inp.reshape(B * N, C_IN),
                         points.reshape(B * N, DIM)], axis=1),
        ((0, 0), (0, _W - C_IN - DIM)))

    w1r = w1.reshape(2 * NC, DIM, NC)
    w1e = jnp.sum(w1r, axis=2).T
    w1p = jnp.pad(w1e, ((C_IN, _W - C_IN - DIM), (0, 0)))
    b1e = b1 - jnp.sum(w1r * centers[None], axis=(1, 2))
    hb = b1e[None] - next_pts.reshape(B * N, DIM) @ w1e
    w2t = w2.T
    b2r = b2.reshape(1, NC)
    w3t = w3.T
    b3r = b3.reshape(1, NC)
    wn = (jnp.transpose(weight, (1, 0, 2)) / K).reshape(NC * C_IN, C_OUT)
    bias_r = bias.reshape(1, C_OUT)

    NCH = 4
    P = 1024
    pts_per_chunk = (B * N) // NCH
    e_per_chunk = pts_per_chunk * K
    nb = pts_per_chunk // P
    body = functools.partial(_tc_body, kk=K, nc=NC, c=C_IN)
    outs = []
    for ci in range(NCH):
        g = _sc_gather(tbl, idx_flat, ci * e_per_chunk, e_per_chunk, 256)
        base = ci * nb
        tc_call = pl.pallas_call(
            body,
            grid=(nb,),
            in_specs=[
                pl.BlockSpec((P * K, _W), lambda i: (i, 0)),
                pl.BlockSpec((P, 2 * NC), lambda i, b=base: (b + i, 0)),
                pl.BlockSpec((_W, 2 * NC), lambda i: (0, 0)),
                pl.BlockSpec((2 * NC, NC), lambda i: (0, 0)),
                pl.BlockSpec((1, NC), lambda i: (0, 0)),
                pl.BlockSpec((NC, NC), lambda i: (0, 0)),
                pl.BlockSpec((1, NC), lambda i: (0, 0)),
                pl.BlockSpec((C_IN * NC, C_OUT), lambda i: (0, 0)),
                pl.BlockSpec((1, C_OUT), lambda i: (0, 0)),
            ],
            out_specs=pl.BlockSpec((P, C_OUT), lambda i: (i, 0)),
            out_shape=jax.ShapeDtypeStruct((pts_per_chunk, C_OUT), jnp.float32),
        )
        outs.append(tc_call(g, hb, w1p, w2t, b2r, w3t, b3r, wn, bias_r))
    out = jnp.concatenate(outs, axis=0)

    return out.reshape(B, N, C_OUT)


# speedup vs baseline: 2.2374x; 1.0231x over previous
import functools

import jax
import jax.numpy as jnp
from jax import lax
from jax.experimental import pallas as pl
from jax.experimental.pallas import tpu as pltpu
from jax.experimental.pallas import tpu_sc as plsc

_W = 128


def _sc_gather(tbl, idx_full, e0, e_chunk, chunk):
    info = plsc.get_sparse_core_info()
    ncores, nsub = info.num_cores, info.num_subcores
    nw = ncores * nsub
    per_w = e_chunk // nw
    n_pairs = per_w // (2 * chunk)

    mesh = plsc.VectorSubcoreMesh(core_axis_name="c", subcore_axis_name="s")

    @functools.partial(
        pl.kernel,
        mesh=mesh,
        compiler_params=pltpu.CompilerParams(use_tc_tiling_on_sc=True),
        out_type=jax.ShapeDtypeStruct((e_chunk, _W), jnp.float32),
        scratch_types=[
            pltpu.VMEM((chunk,), jnp.int32),
            pltpu.VMEM((chunk,), jnp.int32),
            pltpu.VMEM((chunk, _W), jnp.float32),
            pltpu.VMEM((chunk, _W), jnp.float32),
            pltpu.SemaphoreType.DMA,
            pltpu.SemaphoreType.DMA,
            pltpu.SemaphoreType.DMA,
            pltpu.SemaphoreType.DMA,
        ],
    )
    def gather_kernel(tbl_hbm, idx_hbm, out_hbm, idx_v0, idx_v1,
                      row_v0, row_v1, gsem0, gsem1, wsem0, wsem1):
        wid = lax.axis_index("s") * ncores + lax.axis_index("c")
        base = wid * per_w

        def body(i, carry):
            off0 = base + i * 2 * chunk
            off1 = off0 + chunk
            pltpu.sync_copy(idx_hbm.at[pl.ds(e0 + off0, chunk)], idx_v0)
            g0 = pltpu.async_copy(tbl_hbm.at[idx_v0], row_v0, gsem0)
            pltpu.sync_copy(idx_hbm.at[pl.ds(e0 + off1, chunk)], idx_v1)
            g1 = pltpu.async_copy(tbl_hbm.at[idx_v1], row_v1, gsem1)
            g0.wait()
            w0 = pltpu.async_copy(row_v0, out_hbm.at[pl.ds(off0, chunk)],
                                  wsem0)
            g1.wait()
            w1 = pltpu.async_copy(row_v1, out_hbm.at[pl.ds(off1, chunk)],
                                  wsem1)
            w0.wait()
            w1.wait()
            return carry

        lax.fori_loop(0, n_pairs, body, 0)

    return gather_kernel(tbl, idx_full)


def _tc_body(g_ref, hb_ref, w1_ref, w2_ref, b2_ref,
             w3_ref, b3_ref, wn_ref, bias_ref, out_ref, *, kk, nc, c):
    p = hb_ref.shape[0]
    g = g_ref[...]
    h = jnp.dot(g, w1_ref[...], preferred_element_type=jnp.float32)
    h = h.reshape(p, kk, 2 * nc) + hb_ref[...][:, None, :]
    h = jnp.maximum(h, 0.0).reshape(p * kk, 2 * nc)
    h = jnp.dot(h, w2_ref[...], preferred_element_type=jnp.float32)
    h = jnp.maximum(h + b2_ref[...], 0.0)
    d = jnp.dot(h, w3_ref[...], preferred_element_type=jnp.float32)
    d = jnp.maximum(d + b3_ref[...], 0.0)
    d3 = d.reshape(p, kk, nc)
    feat3 = g.reshape(p, kk, _W)[:, :, :c]
    fpre = lax.dot_general(d3, feat3, (((1,), (1,)), ((0,), (0,))),
                           preferred_element_type=jnp.float32)
    out_ref[...] = jnp.dot(fpre.reshape(p, nc * c), wn_ref[...],
                           preferred_element_type=jnp.float32) + bias_ref[...]


def kernel(inp, points, next_pts, indices_, K, weight, bias, centers,
           w1, b1, w2, b2, w3, b3):
    B, N, C_IN = inp.shape
    DIM = points.shape[2]
    NC = centers.shape[1]
    C_OUT = weight.shape[2]
    K = indices_.shape[2]
    E = B * N * K

    offs = (jnp.arange(B, dtype=jnp.int32) * N)[:, None, None]
    idx_flat = (indices_.astype(jnp.int32) + offs).reshape(E)
    tbl = jnp.pad(
        jnp.concatenate([inp.reshape(B * N, C_IN),
                         points.reshape(B * N, DIM)], axis=1),
        ((0, 0), (0, _W - C_IN - DIM)))

    w1r = w1.reshape(2 * NC, DIM, NC)
    w1e = jnp.sum(w1r, axis=2).T
    w1p = jnp.pad(w1e, ((C_IN, _W - C_IN - DIM), (0, 0)))
    b1e = b1 - jnp.sum(w1r * centers[None], axis=(1, 2))
    hb = b1e[None] - next_pts.reshape(B * N, DIM) @ w1e
    w2t = w2.T
    b2r = b2.reshape(1, NC)
    w3t = w3.T
    b3r = b3.reshape(1, NC)
    wn = (jnp.transpose(weight, (1, 0, 2)) / K).reshape(NC * C_IN, C_OUT)
    bias_r = bias.reshape(1, C_OUT)

    NCH = 8
    P = 1024
    pts_per_chunk = (B * N) // NCH
    e_per_chunk = pts_per_chunk * K
    nb = pts_per_chunk // P
    body = functools.partial(_tc_body, kk=K, nc=NC, c=C_IN)
    outs = []
    for ci in range(NCH):
        g = _sc_gather(tbl, idx_flat, ci * e_per_chunk, e_per_chunk, 256)
        base = ci * nb
        tc_call = pl.pallas_call(
            body,
            grid=(nb,),
            in_specs=[
                pl.BlockSpec((P * K, _W), lambda i: (i, 0)),
                pl.BlockSpec((P, 2 * NC), lambda i, b=base: (b + i, 0)),
                pl.BlockSpec((_W, 2 * NC), lambda i: (0, 0)),
                pl.BlockSpec((2 * NC, NC), lambda i: (0, 0)),
                pl.BlockSpec((1, NC), lambda i: (0, 0)),
                pl.BlockSpec((NC, NC), lambda i: (0, 0)),
                pl.BlockSpec((1, NC), lambda i: (0, 0)),
                pl.BlockSpec((C_IN * NC, C_OUT), lambda i: (0, 0)),
                pl.BlockSpec((1, C_OUT), lambda i: (0, 0)),
            ],
            out_specs=pl.BlockSpec((P, C_OUT), lambda i: (i, 0)),
            out_shape=jax.ShapeDtypeStruct((pts_per_chunk, C_OUT), jnp.float32),
        )
        outs.append(tc_call(g, hb, w1p, w2t, b2r, w3t, b3r, wn, bias_r))
    out = jnp.concatenate(outs, axis=0)

    return out.reshape(B, N, C_OUT)
